# baseline (speedup 1.0000x reference)
import functools

import jax
import jax.numpy as jnp
from jax import lax
from jax.experimental import pallas as pl
from jax.experimental.pallas import tpu as pltpu
from jax.experimental.pallas import tpu_sc as plsc

NC = 2
NS = 16
NW = NC * NS

G = 16
NBUF = 4
CH0 = 128


@functools.lru_cache(maxsize=None)
def _make_pool(B, S, D, V):
    assert B % (NW * G) == 0 and S % 8 == 0 and D % 16 == 0
    EPW = B // NW
    NGRP = EPW // G
    CH1 = S - CH0
    mesh = plsc.VectorSubcoreMesh(
        core_axis_name="c", subcore_axis_name="s",
        num_cores=NC, num_subcores=NS)

    @functools.partial(
        pl.kernel,
        out_type=jax.ShapeDtypeStruct((B, D), jnp.float32),
        mesh=mesh,
        compiler_params=pltpu.CompilerParams(
            use_tc_tiling_on_sc=False, needs_layout_passes=False),
        scratch_types=[
            pltpu.VMEM((2, G, S), jnp.int32),
            pltpu.VMEM((NBUF, S, D // 2), jnp.int32),
            pltpu.VMEM((2, G, D), jnp.float32),
            pltpu.SemaphoreType.DMA,
            pltpu.SemaphoreType.DMA,
            pltpu.SemaphoreType.DMA,
            pltpu.SemaphoreType.DMA,
            pltpu.SemaphoreType.DMA,
            pltpu.SemaphoreType.DMA,
        ],
    )
    def pool(x_hbm, table_hbm, out_hbm, idxb, rows, outb,
             g0, g1, g2, g3, isem, osem):
        gsems = (g0, g1, g2, g3)
        wid = lax.axis_index("s") * NC + lax.axis_index("c")
        base = wid * EPW

        def gather_pair(ig, e, j):
            c0 = pltpu.make_async_copy(
                table_hbm.at[idxb.at[ig, e, pl.ds(0, CH0)]],
                rows.at[j, pl.ds(0, CH0)], gsems[j])
            c1 = pltpu.make_async_copy(
                table_hbm.at[idxb.at[ig, e, pl.ds(CH0, CH1)]],
                rows.at[j, pl.ds(CH0, CH1)], gsems[j])
            return c0, c1

        def reduce_rows(j):
            zero = jnp.zeros((16,), jnp.float32)

            def body(m, accs):
                accs = list(accs)
                for r in range(4):
                    p = r % 2
                    for c in range(2):
                        v = rows[j, m * 4 + r, pl.ds(c * 16, 16)]
                        lo = plsc.bitcast(
                            jax.lax.shift_left(v, jnp.int32(16)),
                            jnp.float32)
                        hi = plsc.bitcast(v, jnp.float32)
                        k = p * 8 + c * 2
                        accs[k] = accs[k] + lo
                        accs[k + 1] = accs[k + 1] + hi
                return tuple(accs)

            accs = lax.fori_loop(0, S // 4, body, (zero,) * 16)
            return [accs[k] + accs[8 + k] for k in range(4)]

        def out_copy(og, g):
            return pltpu.make_async_copy(
                outb.at[og], out_hbm.at[pl.ds(base + g * G, G)], osem)

        def idx_copy(ig, g):
            return pltpu.make_async_copy(
                x_hbm.at[pl.ds(base + g * G, G)], idxb.at[ig], isem)

        pltpu.sync_copy(x_hbm.at[pl.ds(base, G)], idxb.at[0])

        def gbody(g, _):
            ig = lax.rem(g, 2)

            @pl.when(g >= 2)
            def _():
                out_copy(ig, g - 2).wait()

            @pl.when(g + 1 < NGRP)
            def _():
                idx_copy(1 - ig, g + 1).start()

            for j in range(NBUF):
                c0, c1 = gather_pair(ig, j, j)
                c0.start()
                c1.start()

            def inner(k, _):
                for j in range(NBUF):
                    e = k * NBUF + j
                    c0, c1 = gather_pair(ig, e, j)
                    c0.wait()
                    c1.wait()
                    vecs = reduce_rows(j)
                    for kk in range(4):
                        outb[ig, e, pl.ds(kk * 16, 16)] = vecs[kk]

                    @pl.when(k < G // NBUF - 1)
                    def _():
                        n0, n1 = gather_pair(ig, e + NBUF, j)
                        n0.start()
                        n1.start()
                return 0

            lax.fori_loop(0, G // NBUF, inner, 0)
            out_copy(ig, g).start()

            @pl.when(g + 1 < NGRP)
            def _():
                idx_copy(1 - ig, g + 1).wait()

            return 0

        lax.fori_loop(0, NGRP, gbody, 0)
        for gg in (NGRP - 2, NGRP - 1):
            out_copy(gg % 2, gg).wait()

    return pool


def _repack_body(*refs):
    stripe_refs, o_ref = refs[:-1], refs[-1]
    x = jnp.concatenate([r[:] for r in stripe_refs], axis=0)
    cb = x.shape[1]
    row = jax.lax.broadcasted_iota(jnp.int32, (128, 128), 0)
    col = jax.lax.broadcasted_iota(jnp.int32, (128, 128), 1)
    sel = jnp.float32(1.0) * (col == 4 * (row % 32) + row // 32)
    for g in range(cb // 128):
        y = x[:, 128 * g:128 * (g + 1)]
        z = jax.lax.dot_general(sel, y, (((1,), (1,)), ((), ())),
                                preferred_element_type=jnp.float32)
        u16 = jax.lax.bitcast_convert_type(
            z.astype(jnp.bfloat16), jnp.uint16)
        u = (u16[:, 0:32].astype(jnp.int32)
             | (u16[:, 32:64].astype(jnp.int32) << 16))
        for q in range(4):
            o_ref[32 * g:32 * (g + 1), 32 * q:32 * (q + 1)] = (
                u[32 * q:32 * (q + 1), :])


@functools.lru_cache(maxsize=None)
def _make_repack(V, D):
    CB = 4096
    grid = ((V + CB - 1) // CB,)

    def stripe_spec(tr):
        return pl.BlockSpec((8, CB), lambda i, tr=tr: (tr, i))

    return pl.pallas_call(
        _repack_body,
        grid=grid,
        in_specs=[stripe_spec(tr) for tr in range(D // 8)],
        out_specs=pl.BlockSpec((CB // 4, 128), lambda i: (i, 0)),
        out_shape=jax.ShapeDtypeStruct((V // 4, 128), jnp.int32),
    )


def _mlp_body(x_ref, w1_ref, b1_ref, w2_ref, b2_ref, o_ref):
    h = jnp.dot(x_ref[:], w1_ref[:], preferred_element_type=jnp.float32)
    h = jnp.maximum(h + b1_ref[:], 0.0)
    logits = jnp.dot(h, w2_ref[:], preferred_element_type=jnp.float32)
    logits = logits + b2_ref[:]
    m = jnp.max(logits, axis=1, keepdims=True)
    ex = jnp.exp(logits - m)
    lse = jnp.log(jnp.sum(ex, axis=1, keepdims=True)) + m
    o_ref[:] = logits - lse


@functools.lru_cache(maxsize=None)
def _make_mlp(B, D, HP):
    BB = 1024
    grid = (B // BB,)
    return pl.pallas_call(
        _mlp_body,
        grid=grid,
        in_specs=[
            pl.BlockSpec((BB, D), lambda i: (i, 0)),
            pl.BlockSpec((D, HP), lambda i: (0, 0)),
            pl.BlockSpec((1, HP), lambda i: (0, 0)),
            pl.BlockSpec((HP, 128), lambda i: (0, 0)),
            pl.BlockSpec((1, 128), lambda i: (0, 0)),
        ],
        out_specs=pl.BlockSpec((BB, 128), lambda i: (i, 0)),
        out_shape=jax.ShapeDtypeStruct((B, 128), jnp.float32),
    )


def kernel(x, table, W1, b1, W2, b2):
    B, S = x.shape
    V, D = table.shape
    H = W1.shape[1]
    HP = (H + 7) // 8 * 8

    table_t = table.T
    table_lin = _make_repack(V, D)(*([table_t] * (D // 8)))
    sums = _make_pool(B, S, D, V)(x, table_lin.reshape(V, D // 2))

    lane = jnp.arange(16)
    perm = jnp.concatenate([lane, 32 + lane, 16 + lane, 48 + lane])
    W1 = W1[perm, :]

    w1 = jnp.pad(W1 * (1.0 / S), ((0, 0), (0, HP - H)))
    bb1 = jnp.pad(b1, (0, HP - H)).reshape(1, HP)
    w2 = jnp.pad(W2, ((0, HP - H), (0, 128 - W2.shape[1])))
    bb2 = jnp.concatenate(
        [b2, jnp.full((128 - b2.shape[0],), -1e9, jnp.float32)]).reshape(1, 128)

    out = _make_mlp(B, D, HP)(sums, w1, bb1, w2, bb2)
    return out[:, : b2.shape[0]]


# speedup vs baseline: 1.0422x; 1.0422x over previous
import functools

import jax
import jax.numpy as jnp
from jax import lax
from jax.experimental import pallas as pl
from jax.experimental.pallas import tpu as pltpu
from jax.experimental.pallas import tpu_sc as plsc

NC = 2
NS = 16
NW = NC * NS

G = 16
NBUF = 4
CH0 = 128


@functools.lru_cache(maxsize=None)
def _make_pool(B, S, D, V):
    assert B % (NW * G) == 0 and S % 8 == 0 and D % 16 == 0
    EPW = B // NW
    NGRP = EPW // G
    CH1 = S - CH0
    mesh = plsc.VectorSubcoreMesh(
        core_axis_name="c", subcore_axis_name="s",
        num_cores=NC, num_subcores=NS)

    @functools.partial(
        pl.kernel,
        out_type=jax.ShapeDtypeStruct((B, D), jnp.float32),
        mesh=mesh,
        compiler_params=pltpu.CompilerParams(
            use_tc_tiling_on_sc=False, needs_layout_passes=False),
        scratch_types=[
            pltpu.VMEM((2, G, S), jnp.int32),
            pltpu.VMEM((NBUF, S, D // 2), jnp.int32),
            pltpu.VMEM((2, G, D), jnp.float32),
            pltpu.SemaphoreType.DMA,
            pltpu.SemaphoreType.DMA,
            pltpu.SemaphoreType.DMA,
            pltpu.SemaphoreType.DMA,
            pltpu.SemaphoreType.DMA,
            pltpu.SemaphoreType.DMA,
        ],
    )
    def pool(x_hbm, table_hbm, out_hbm, idxb, rows, outb,
             g0, g1, g2, g3, isem, osem):
        gsems = (g0, g1, g2, g3)
        wid = lax.axis_index("s") * NC + lax.axis_index("c")
        base = wid * EPW

        def gather_pair(ig, e, j):
            c0 = pltpu.make_async_copy(
                table_hbm.at[idxb.at[ig, e, pl.ds(0, CH0)]],
                rows.at[j, pl.ds(0, CH0)], gsems[j])
            c1 = pltpu.make_async_copy(
                table_hbm.at[idxb.at[ig, e, pl.ds(CH0, CH1)]],
                rows.at[j, pl.ds(CH0, CH1)], gsems[j])
            return c0, c1

        def reduce_rows(j):
            zero = jnp.zeros((16,), jnp.float32)

            def body(m, accs):
                accs = list(accs)
                for r in range(4):
                    p = r % 2
                    for c in range(2):
                        v = rows[j, m * 4 + r, pl.ds(c * 16, 16)]
                        lo = plsc.bitcast(
                            jax.lax.shift_left(v, jnp.int32(16)),
                            jnp.float32)
                        hi = plsc.bitcast(v, jnp.float32)
                        k = p * 8 + c * 2
                        accs[k] = accs[k] + lo
                        accs[k + 1] = accs[k + 1] + hi
                return tuple(accs)

            accs = lax.fori_loop(0, S // 4, body, (zero,) * 16)
            return [accs[k] + accs[8 + k] for k in range(4)]

        def out_copy(og, g):
            return pltpu.make_async_copy(
                outb.at[og], out_hbm.at[pl.ds(base + g * G, G)], osem)

        def idx_copy(ig, g):
            return pltpu.make_async_copy(
                x_hbm.at[pl.ds(base + g * G, G)], idxb.at[ig], isem)

        pltpu.sync_copy(x_hbm.at[pl.ds(base, G)], idxb.at[0])

        def gbody(g, _):
            ig = lax.rem(g, 2)

            @pl.when(g >= 2)
            def _():
                out_copy(ig, g - 2).wait()

            @pl.when(g + 1 < NGRP)
            def _():
                idx_copy(1 - ig, g + 1).start()

            for j in range(NBUF):
                c0, c1 = gather_pair(ig, j, j)
                c0.start()
                c1.start()

            def inner(k, _):
                for j in range(NBUF):
                    e = k * NBUF + j
                    c0, c1 = gather_pair(ig, e, j)
                    c0.wait()
                    c1.wait()
                    vecs = reduce_rows(j)
                    for kk in range(4):
                        outb[ig, e, pl.ds(kk * 16, 16)] = vecs[kk]

                    @pl.when(k < G // NBUF - 1)
                    def _():
                        n0, n1 = gather_pair(ig, e + NBUF, j)
                        n0.start()
                        n1.start()
                return 0

            lax.fori_loop(0, G // NBUF, inner, 0)
            out_copy(ig, g).start()

            @pl.when(g + 1 < NGRP)
            def _():
                idx_copy(1 - ig, g + 1).wait()

            return 0

        lax.fori_loop(0, NGRP, gbody, 0)
        for gg in (NGRP - 2, NGRP - 1):
            out_copy(gg % 2, gg).wait()

    return pool


def _repack_body(*refs):
    stripe_refs, o_ref = refs[:-1], refs[-1]
    x = jnp.concatenate([r[:] for r in stripe_refs], axis=0)
    cb = x.shape[1]
    row = jax.lax.broadcasted_iota(jnp.int32, (128, 128), 0)
    col = jax.lax.broadcasted_iota(jnp.int32, (128, 128), 1)
    sel = jnp.float32(1.0) * (col == 4 * (row % 32) + row // 32)
    for g in range(cb // 128):
        y = x[:, 128 * g:128 * (g + 1)]
        z = jax.lax.dot_general(sel, y, (((1,), (1,)), ((), ())),
                                preferred_element_type=jnp.float32)
        u16 = jax.lax.bitcast_convert_type(
            z.astype(jnp.bfloat16), jnp.uint16)
        u = (u16[:, 0:32].astype(jnp.int32)
             | (u16[:, 32:64].astype(jnp.int32) << 16))
        for q in range(4):
            o_ref[32 * g:32 * (g + 1), 32 * q:32 * (q + 1)] = (
                u[32 * q:32 * (q + 1), :])


@functools.lru_cache(maxsize=None)
def _make_repack(V, D):
    CB = 2048
    grid = ((V + CB - 1) // CB,)

    def stripe_spec(tr):
        return pl.BlockSpec((8, CB), lambda i, tr=tr: (tr, i))

    return pl.pallas_call(
        _repack_body,
        grid=grid,
        in_specs=[stripe_spec(tr) for tr in range(D // 8)],
        out_specs=pl.BlockSpec((CB // 4, 128), lambda i: (i, 0)),
        out_shape=jax.ShapeDtypeStruct((V // 4, 128), jnp.int32),
    )


def _mlp_body(x_ref, w1_ref, b1_ref, w2_ref, b2_ref, o_ref):
    h = jnp.dot(x_ref[:], w1_ref[:], preferred_element_type=jnp.float32)
    h = jnp.maximum(h + b1_ref[:], 0.0)
    logits = jnp.dot(h, w2_ref[:], preferred_element_type=jnp.float32)
    logits = logits + b2_ref[:]
    m = jnp.max(logits, axis=1, keepdims=True)
    ex = jnp.exp(logits - m)
    lse = jnp.log(jnp.sum(ex, axis=1, keepdims=True)) + m
    o_ref[:] = logits - lse


@functools.lru_cache(maxsize=None)
def _make_mlp(B, D, HP):
    BB = 1024
    grid = (B // BB,)
    return pl.pallas_call(
        _mlp_body,
        grid=grid,
        in_specs=[
            pl.BlockSpec((BB, D), lambda i: (i, 0)),
            pl.BlockSpec((D, HP), lambda i: (0, 0)),
            pl.BlockSpec((1, HP), lambda i: (0, 0)),
            pl.BlockSpec((HP, 128), lambda i: (0, 0)),
            pl.BlockSpec((1, 128), lambda i: (0, 0)),
        ],
        out_specs=pl.BlockSpec((BB, 128), lambda i: (i, 0)),
        out_shape=jax.ShapeDtypeStruct((B, 128), jnp.float32),
    )


def kernel(x, table, W1, b1, W2, b2):
    B, S = x.shape
    V, D = table.shape
    H = W1.shape[1]
    HP = (H + 7) // 8 * 8

    table_t = table.T
    table_lin = _make_repack(V, D)(*([table_t] * (D // 8)))
    sums = _make_pool(B, S, D, V)(x, table_lin.reshape(V, D // 2))

    lane = jnp.arange(16)
    perm = jnp.concatenate([lane, 32 + lane, 16 + lane, 48 + lane])
    W1 = W1[perm, :]

    w1 = jnp.pad(W1 * (1.0 / S), ((0, 0), (0, HP - H)))
    bb1 = jnp.pad(b1, (0, HP - H)).reshape(1, HP)
    w2 = jnp.pad(W2, ((0, HP - H), (0, 128 - W2.shape[1])))
    bb2 = jnp.concatenate(
        [b2, jnp.full((128 - b2.shape[0],), -1e9, jnp.float32)]).reshape(1, 128)

    out = _make_mlp(B, D, HP)(sums, w1, bb1, w2, bb2)
    return out[:, : b2.shape[0]]


# speedup vs baseline: 1.0619x; 1.0190x over previous
import functools

import jax
import jax.numpy as jnp
from jax import lax
from jax.experimental import pallas as pl
from jax.experimental.pallas import tpu as pltpu
from jax.experimental.pallas import tpu_sc as plsc

NC = 2
NS = 16
NW = NC * NS

G = 16
NBUF = 8
CH0 = 128


@functools.lru_cache(maxsize=None)
def _make_pool(B, S, D, V):
    assert B % (NW * G) == 0 and S % 8 == 0 and D % 16 == 0
    EPW = B // NW
    NGRP = EPW // G
    CH1 = S - CH0
    mesh = plsc.VectorSubcoreMesh(
        core_axis_name="c", subcore_axis_name="s",
        num_cores=NC, num_subcores=NS)

    @functools.partial(
        pl.kernel,
        out_type=jax.ShapeDtypeStruct((B, D), jnp.float32),
        mesh=mesh,
        compiler_params=pltpu.CompilerParams(
            use_tc_tiling_on_sc=False, needs_layout_passes=False),
        scratch_types=[
            pltpu.VMEM((2, G, S), jnp.int32),
            pltpu.VMEM((NBUF, S, D // 2), jnp.int32),
            pltpu.VMEM((2, G, D), jnp.float32),
        ] + [pltpu.SemaphoreType.DMA] * (NBUF + 2),
    )
    def pool(x_hbm, table_hbm, out_hbm, idxb, rows, outb, *sems):
        gsems = sems[:NBUF]
        isem, osem = sems[NBUF], sems[NBUF + 1]
        wid = lax.axis_index("s") * NC + lax.axis_index("c")
        base = wid * EPW

        def gather_pair(ig, e, j):
            c0 = pltpu.make_async_copy(
                table_hbm.at[idxb.at[ig, e, pl.ds(0, CH0)]],
                rows.at[j, pl.ds(0, CH0)], gsems[j])
            c1 = pltpu.make_async_copy(
                table_hbm.at[idxb.at[ig, e, pl.ds(CH0, CH1)]],
                rows.at[j, pl.ds(CH0, CH1)], gsems[j])
            return c0, c1

        def reduce_rows(j):
            zero = jnp.zeros((16,), jnp.float32)

            def body(m, accs):
                accs = list(accs)
                for r in range(8):
                    p = r % 2
                    for c in range(2):
                        v = rows[j, m * 8 + r, pl.ds(c * 16, 16)]
                        lo = plsc.bitcast(
                            jax.lax.shift_left(v, jnp.int32(16)),
                            jnp.float32)
                        hi = plsc.bitcast(v, jnp.float32)
                        k = p * 8 + c * 2
                        accs[k] = accs[k] + lo
                        accs[k + 1] = accs[k + 1] + hi
                return tuple(accs)

            accs = lax.fori_loop(0, S // 8, body, (zero,) * 16)
            return [accs[k] + accs[8 + k] for k in range(4)]

        def out_copy(og, g):
            return pltpu.make_async_copy(
                outb.at[og], out_hbm.at[pl.ds(base + g * G, G)], osem)

        def idx_copy(ig, g):
            return pltpu.make_async_copy(
                x_hbm.at[pl.ds(base + g * G, G)], idxb.at[ig], isem)

        pltpu.sync_copy(x_hbm.at[pl.ds(base, G)], idxb.at[0])

        def gbody(g, _):
            ig = lax.rem(g, 2)

            @pl.when(g >= 2)
            def _():
                out_copy(ig, g - 2).wait()

            @pl.when(g + 1 < NGRP)
            def _():
                idx_copy(1 - ig, g + 1).start()

            for j in range(NBUF):
                c0, c1 = gather_pair(ig, j, j)
                c0.start()
                c1.start()

            def inner(k, _):
                for j in range(NBUF):
                    e = k * NBUF + j
                    c0, c1 = gather_pair(ig, e, j)
                    c0.wait()
                    c1.wait()
                    vecs = reduce_rows(j)
                    for kk in range(4):
                        outb[ig, e, pl.ds(kk * 16, 16)] = vecs[kk]

                    @pl.when(k < G // NBUF - 1)
                    def _():
                        n0, n1 = gather_pair(ig, e + NBUF, j)
                        n0.start()
                        n1.start()
                return 0

            lax.fori_loop(0, G // NBUF, inner, 0)
            out_copy(ig, g).start()

            @pl.when(g + 1 < NGRP)
            def _():
                idx_copy(1 - ig, g + 1).wait()

            return 0

        lax.fori_loop(0, NGRP, gbody, 0)
        for gg in (NGRP - 2, NGRP - 1):
            out_copy(gg % 2, gg).wait()

    return pool


def _repack_body(*refs):
    stripe_refs, o_ref = refs[:-1], refs[-1]
    x = jnp.concatenate([r[:] for r in stripe_refs], axis=0)
    cb = x.shape[1]
    row = jax.lax.broadcasted_iota(jnp.int32, (128, 128), 0)
    col = jax.lax.broadcasted_iota(jnp.int32, (128, 128), 1)
    sel = jnp.float32(1.0) * (col == 4 * (row % 32) + row // 32)
    for g in range(cb // 128):
        y = x[:, 128 * g:128 * (g + 1)]
        z = jax.lax.dot_general(sel, y, (((1,), (1,)), ((), ())),
                                preferred_element_type=jnp.float32)
        u16 = jax.lax.bitcast_convert_type(
            z.astype(jnp.bfloat16), jnp.uint16)
        u = (u16[:, 0:32].astype(jnp.int32)
             | (u16[:, 32:64].astype(jnp.int32) << 16))
        for q in range(4):
            o_ref[32 * g:32 * (g + 1), 32 * q:32 * (q + 1)] = (
                u[32 * q:32 * (q + 1), :])


@functools.lru_cache(maxsize=None)
def _make_repack(V, D):
    CB = 2048
    grid = ((V + CB - 1) // CB,)

    def stripe_spec(tr):
        return pl.BlockSpec((8, CB), lambda i, tr=tr: (tr, i))

    return pl.pallas_call(
        _repack_body,
        grid=grid,
        in_specs=[stripe_spec(tr) for tr in range(D // 8)],
        out_specs=pl.BlockSpec((CB // 4, 128), lambda i: (i, 0)),
        out_shape=jax.ShapeDtypeStruct((V // 4, 128), jnp.int32),
    )


def _mlp_body(x_ref, w1_ref, b1_ref, w2_ref, b2_ref, o_ref):
    h = jnp.dot(x_ref[:], w1_ref[:], preferred_element_type=jnp.float32)
    h = jnp.maximum(h + b1_ref[:], 0.0)
    logits = jnp.dot(h, w2_ref[:], preferred_element_type=jnp.float32)
    logits = logits + b2_ref[:]
    m = jnp.max(logits, axis=1, keepdims=True)
    ex = jnp.exp(logits - m)
    lse = jnp.log(jnp.sum(ex, axis=1, keepdims=True)) + m
    o_ref[:] = logits - lse


@functools.lru_cache(maxsize=None)
def _make_mlp(B, D, HP):
    BB = 1024
    grid = (B // BB,)
    return pl.pallas_call(
        _mlp_body,
        grid=grid,
        in_specs=[
            pl.BlockSpec((BB, D), lambda i: (i, 0)),
            pl.BlockSpec((D, HP), lambda i: (0, 0)),
            pl.BlockSpec((1, HP), lambda i: (0, 0)),
            pl.BlockSpec((HP, 128), lambda i: (0, 0)),
            pl.BlockSpec((1, 128), lambda i: (0, 0)),
        ],
        out_specs=pl.BlockSpec((BB, 128), lambda i: (i, 0)),
        out_shape=jax.ShapeDtypeStruct((B, 128), jnp.float32),
    )


def kernel(x, table, W1, b1, W2, b2):
    B, S = x.shape
    V, D = table.shape
    H = W1.shape[1]
    HP = (H + 7) // 8 * 8

    table_t = table.T
    table_lin = _make_repack(V, D)(*([table_t] * (D // 8)))
    sums = _make_pool(B, S, D, V)(x, table_lin.reshape(V, D // 2))

    lane = jnp.arange(16)
    perm = jnp.concatenate([lane, 32 + lane, 16 + lane, 48 + lane])
    W1 = W1[perm, :]

    w1 = jnp.pad(W1 * (1.0 / S), ((0, 0), (0, HP - H)))
    bb1 = jnp.pad(b1, (0, HP - H)).reshape(1, HP)
    w2 = jnp.pad(W2, ((0, HP - H), (0, 128 - W2.shape[1])))
    bb2 = jnp.concatenate(
        [b2, jnp.full((128 - b2.shape[0],), -1e9, jnp.float32)]).reshape(1, 128)

    out = _make_mlp(B, D, HP)(sums, w1, bb1, w2, bb2)
    return out[:, : b2.shape[0]]
